# baseline (speedup 1.0000x reference)
import functools

import jax
import jax.numpy as jnp
from jax import lax
from jax.experimental import pallas as pl
from jax.experimental.pallas import tpu as pltpu
from jax.experimental.pallas import tpu_sc as plsc

N = 10000
E = 320000
D = 128
H = 128

ROWS = 1000
NTILES = 16
NCORES = 2
EPT = E // NTILES
K = 80
NCHUNK = EPT // K
EPW = E // (NTILES * NCORES)
NP = 10240
RPT = NP // NTILES


def _k1_body(x_ref, w1_ref, b1_ref, w2_ref, b2_ref, w3_ref, b3_ref, h_ref):
    x = x_ref[...]
    h = jnp.maximum(jnp.dot(x, w1_ref[...], preferred_element_type=jnp.float32) + b1_ref[...], 0.0)
    h = jnp.maximum(jnp.dot(h, w2_ref[...], preferred_element_type=jnp.float32) + b2_ref[...], 0.0)
    h_ref[...] = jnp.dot(h, w3_ref[...], preferred_element_type=jnp.float32) + b3_ref[...]


def _k1(x, W1, b1, W2, b2, W3, b3):
    return pl.pallas_call(
        _k1_body,
        grid=(N // ROWS,),
        in_specs=[
            pl.BlockSpec((ROWS, D), lambda i: (i, 0)),
            pl.BlockSpec((D, H), lambda i: (0, 0)),
            pl.BlockSpec((1, H), lambda i: (0, 0)),
            pl.BlockSpec((H, H), lambda i: (0, 0)),
            pl.BlockSpec((1, H), lambda i: (0, 0)),
            pl.BlockSpec((H, H), lambda i: (0, 0)),
            pl.BlockSpec((1, H), lambda i: (0, 0)),
        ],
        out_specs=pl.BlockSpec((ROWS, H), lambda i: (i, 0)),
        out_shape=jax.ShapeDtypeStruct((N, H), jnp.float32),
    )(x, W1, b1.reshape(1, H), W2, b2.reshape(1, H), W3, b3.reshape(1, H))


def _kdeg_body(dst_hbm, onecol_hbm, zeros_hbm, out_hbm, didx, onecol, deg_sh, sem):
    c = lax.axis_index("c")
    t = lax.axis_index("s")
    wid = c * NTILES + t

    pltpu.sync_copy(onecol_hbm, onecol)
    pltpu.sync_copy(zeros_hbm.at[pl.ds(t * RPT, RPT)], deg_sh.at[pl.ds(t * RPT, RPT)])
    plsc.subcore_barrier()

    def chunk(i, _):
        pltpu.sync_copy(dst_hbm.at[pl.ds(wid * EPW + i * K, K)], didx)
        pltpu.sync_copy(onecol, deg_sh.at[didx], add=True)
        return _
    lax.fori_loop(0, EPW // K, chunk, None)

    plsc.subcore_barrier()
    pltpu.sync_copy(deg_sh.at[pl.ds(t * RPT, RPT)], out_hbm.at[c, pl.ds(t * RPT, RPT)])


def _kdeg(dst):
    mesh = plsc.VectorSubcoreMesh(core_axis_name="c", subcore_axis_name="s")
    f = pl.kernel(
        _kdeg_body,
        out_type=jax.ShapeDtypeStruct((NCORES, NP, 16), jnp.float32),
        mesh=mesh,
        compiler_params=pltpu.CompilerParams(use_tc_tiling_on_sc=False, needs_layout_passes=False),
        scratch_types=[
            pltpu.VMEM((K,), jnp.int32),
            pltpu.VMEM((K, 16), jnp.float32),
            pltpu.VMEM_SHARED((NP, 16), jnp.float32),
            pltpu.SemaphoreType.DMA,
        ],
    )
    onecol = jnp.tile(jnp.eye(1, 16, dtype=jnp.float32), (K, 1))
    zeros = jnp.zeros((NP, 16), jnp.float32)
    return f(dst, onecol, zeros)


def _k1b_body(degp_ref, h_ref, dinv_ref, u_ref, hn_ref, inv_ref):
    deg = 1.0 + jnp.sum(degp_ref[0], axis=0)
    dinv = jax.lax.rsqrt(jnp.maximum(deg, 1e-12))
    dinv_ref[...] = dinv.reshape(1, 1, ROWS)
    h = h_ref[...]
    u_ref[...] = dinv[:, None] * h
    n2 = jnp.sum(h * h, axis=1)
    invn = jax.lax.rsqrt(jnp.maximum(n2, 1e-24))
    hn_ref[...] = invn[:, None] * h
    inv_ref[...] = invn.reshape(1, 1, ROWS)


def _k1b(degp_t, h):
    return pl.pallas_call(
        _k1b_body,
        grid=(N // ROWS,),
        in_specs=[
            pl.BlockSpec((1, NCORES, ROWS), lambda i: (i, 0, 0)),
            pl.BlockSpec((ROWS, H), lambda i: (i, 0)),
        ],
        out_specs=[
            pl.BlockSpec((1, 1, ROWS), lambda i: (i, 0, 0)),
            pl.BlockSpec((ROWS, H), lambda i: (i, 0)),
            pl.BlockSpec((ROWS, H), lambda i: (i, 0)),
            pl.BlockSpec((1, 1, ROWS), lambda i: (i, 0, 0)),
        ],
        out_shape=[
            jax.ShapeDtypeStruct((N // ROWS, 1, ROWS), jnp.float32),
            jax.ShapeDtypeStruct((N, H), jnp.float32),
            jax.ShapeDtypeStruct((N, H), jnp.float32),
            jax.ShapeDtypeStruct((N // ROWS, 1, ROWS), jnp.float32),
        ],
    )(degp_t, h)


def _kedge_body(h_hbm, hn_hbm, u_hbm, inv_hbm, src_hbm, dst_hbm, beta_hbm,
                zrows_hbm, zden_hbm,
                out_hbm, den_hbm,
                betav, sidx, didx, rows_s, pay, svals, inv_v,
                acc, den_sh, sem_s, sem_d):
    role = lax.axis_index("c")
    t = lax.axis_index("s")
    zero16 = jnp.zeros((16,), jnp.float32)
    iota16 = lax.iota(jnp.int32, 16)
    e16s = [g * 16 + iota16 for g in range(K // 16)]

    pltpu.sync_copy(beta_hbm, betav)
    pltpu.sync_copy(inv_hbm, inv_v)
    pltpu.sync_copy(zrows_hbm.at[pl.ds(t * RPT, RPT)], acc.at[pl.ds(t * RPT, RPT)])
    pltpu.sync_copy(zden_hbm.at[pl.ds(t * RPT, RPT)], den_sh.at[pl.ds(t * RPT, RPT)])

    def zcol(cc, _):
        j16 = jnp.full((16,), cc, jnp.int32) + 1
        for g in range(K // 16):
            plsc.store_scatter(svals, [e16s[g], j16], zero16)
        return _
    lax.fori_loop(0, 15, zcol, None)

    plsc.subcore_barrier()

    def chunk(i, _):
        base = t * EPT + i * K
        pltpu.sync_copy(src_hbm.at[pl.ds(base, K)], sidx)
        pltpu.sync_copy(dst_hbm.at[pl.ds(base, K)], didx)

        @pl.when(role == 0)
        def _r0():
            pltpu.async_copy(u_hbm.at[sidx], rows_s, sem_s).wait()
            pltpu.sync_copy(rows_s, acc.at[didx], add=True)

        @pl.when(role == 1)
        def _r1():
            pltpu.async_copy(h_hbm.at[sidx], rows_s, sem_s).wait()
            pltpu.async_copy(hn_hbm.at[didx], pay, sem_d).wait()

            for g in range(K // 16):
                e16 = e16s[g]

                def dj(jblk, q):
                    for jj in range(8):
                        j16 = jnp.full((16,), jblk * 8 + jj, jnp.int32)
                        va = plsc.load_gather(rows_s, [e16, j16])
                        vb = plsc.load_gather(pay, [e16, j16])
                        q = q + va * vb
                    return q
                q = lax.fori_loop(0, D // 8, dj, zero16)
                inv16 = plsc.load_gather(inv_v, [sidx[pl.ds(g * 16, 16)]])
                s16 = jnp.exp(betav[...] * (q * inv16))
                plsc.store_scatter(svals, [e16, jnp.zeros((16,), jnp.int32)], s16)

                def pj(jblk, _2):
                    for jj in range(8):
                        j16 = jnp.full((16,), jblk * 8 + jj, jnp.int32)
                        va = plsc.load_gather(rows_s, [e16, j16])
                        plsc.store_scatter(pay, [e16, j16], s16 * va)
                    return _2
                lax.fori_loop(0, D // 8, pj, None)

            pltpu.sync_copy(pay, acc.at[didx], add=True)
            pltpu.sync_copy(svals, den_sh.at[didx], add=True)
        return _

    lax.fori_loop(0, NCHUNK, chunk, None)

    plsc.subcore_barrier()
    pltpu.sync_copy(acc.at[pl.ds(t * RPT, RPT)], out_hbm.at[role, pl.ds(t * RPT, RPT)])

    @pl.when(role == 1)
    def _fd():
        pltpu.sync_copy(den_sh.at[pl.ds(t * RPT, RPT)], den_hbm.at[pl.ds(t * RPT, RPT)])


def _kedge(h, hn, u, invn, src, dst, beta16):
    mesh = plsc.VectorSubcoreMesh(core_axis_name="c", subcore_axis_name="s")
    f = pl.kernel(
        _kedge_body,
        out_type=[
            jax.ShapeDtypeStruct((NCORES, NP, D), jnp.float32),
            jax.ShapeDtypeStruct((NP, 16), jnp.float32),
        ],
        mesh=mesh,
        compiler_params=pltpu.CompilerParams(use_tc_tiling_on_sc=False, needs_layout_passes=False),
        scratch_types=[
            pltpu.VMEM((16,), jnp.float32),
            pltpu.VMEM((K,), jnp.int32),
            pltpu.VMEM((K,), jnp.int32),
            pltpu.VMEM((K, D), jnp.float32),
            pltpu.VMEM((K, D), jnp.float32),
            pltpu.VMEM((K, 16), jnp.float32),
            pltpu.VMEM((NP,), jnp.float32),
            pltpu.VMEM_SHARED((NP, D), jnp.float32),
            pltpu.VMEM_SHARED((NP, 16), jnp.float32),
            pltpu.SemaphoreType.DMA,
            pltpu.SemaphoreType.DMA,
        ],
    )
    zrows = jnp.zeros((NP, D), jnp.float32)
    zden = jnp.zeros((NP, 16), jnp.float32)
    return f(h, hn, u, invn, src, dst, beta16, zrows, zden)


def _k2_body(h_ref, gs_ref, as_ref, den_ref, dinv_ref, beta_ref,
             wg1_ref, bg1_ref, wg2_ref, bg2_ref, wf_ref, bf_ref,
             wx_ref, bx_ref, wc1_ref, wc2_ref, bc_ref, out_ref):
    h = h_ref[...]
    dinv = dinv_ref[0, 0, :][:, None]
    beta = beta_ref[0, 0]
    sself = jnp.exp(beta)

    agg = dinv * gs_ref[...] + (dinv * dinv) * h
    h_a = jnp.dot(agg, wg1_ref[...], preferred_element_type=jnp.float32) + bg1_ref[...]
    h_b = jnp.dot(agg, wg2_ref[...], preferred_element_type=jnp.float32) + bg2_ref[...]
    h1 = (as_ref[...] + sself * h) / (den_ref[0, 0, :][:, None] + sself)

    pa = jnp.tanh(jnp.dot(h_a, wf_ref[...], preferred_element_type=jnp.float32) + bf_ref[...])
    pb = jnp.tanh(jnp.dot(h_b, wf_ref[...], preferred_element_type=jnp.float32) + bf_ref[...])
    xp = jnp.tanh(jnp.dot(h, wx_ref[...], preferred_element_type=jnp.float32) + bx_ref[...])
    la = jnp.sum(pa * xp, axis=1)
    lb = jnp.sum(pb * xp, axis=1)
    m = jnp.maximum(la, lb)
    wa = jnp.exp(la - m)
    wb = jnp.exp(lb - m)
    res = (h_a * wa[:, None] + h_b * wb[:, None]) / (wa + wb)[:, None]
    out_ref[...] = (jnp.dot(res, wc1_ref[...], preferred_element_type=jnp.float32)
                    + jnp.dot(h1, wc2_ref[...], preferred_element_type=jnp.float32)
                    + bc_ref[...])


def _k2(h, gcn_sum, agnn_sum, denT, dinvT, beta11,
        Wg1, bg1, Wg2, bg2, Wf, bf, Wx, bx, Wc1p, Wc2p, bcp):
    full = lambda i: (0, 0)
    return pl.pallas_call(
        _k2_body,
        grid=(N // ROWS,),
        in_specs=[
            pl.BlockSpec((ROWS, H), lambda i: (i, 0)),
            pl.BlockSpec((ROWS, H), lambda i: (i, 0)),
            pl.BlockSpec((ROWS, H), lambda i: (i, 0)),
            pl.BlockSpec((1, 1, ROWS), lambda i: (i, 0, 0)),
            pl.BlockSpec((1, 1, ROWS), lambda i: (i, 0, 0)),
            pl.BlockSpec((1, 1), full),
            pl.BlockSpec((H, H), full),
            pl.BlockSpec((1, H), full),
            pl.BlockSpec((H, H), full),
            pl.BlockSpec((1, H), full),
            pl.BlockSpec((H, H), full),
            pl.BlockSpec((1, H), full),
            pl.BlockSpec((H, H), full),
            pl.BlockSpec((1, H), full),
            pl.BlockSpec((H, 128), full),
            pl.BlockSpec((H, 128), full),
            pl.BlockSpec((1, 128), full),
        ],
        out_specs=pl.BlockSpec((ROWS, 128), lambda i: (i, 0)),
        out_shape=jax.ShapeDtypeStruct((N, 128), jnp.float32),
    )(h, gcn_sum, agnn_sum, denT, dinvT, beta11,
      Wg1, bg1.reshape(1, H), Wg2, bg2.reshape(1, H), Wf, bf.reshape(1, H),
      Wx, bx.reshape(1, H), Wc1p, Wc2p, bcp)


def kernel(x, edge_index, W1, b1, W2, b2, W3, b3, Wg1, bg1, Wg2, bg2, beta, Wf, bf, Wx, bx, Wc, bc):
    src = edge_index[0].astype(jnp.int32)
    dst = edge_index[1].astype(jnp.int32)

    h = _k1(x, W1, b1, W2, b2, W3, b3)
    degp = _kdeg(dst)[:, :N, 0]
    degp_t = degp.reshape(NCORES, N // ROWS, ROWS).transpose(1, 0, 2)
    dinvT, u, hn, invT = _k1b(degp_t, h)
    invn = jnp.pad(invT.reshape(-1), (0, NP - N))

    beta16 = jnp.full((16,), beta, jnp.float32)
    acc, den = _kedge(h, hn, u, invn, src, dst, beta16)

    gcn_sum = acc[0, :N]
    agnn_sum = acc[1, :N]
    denT = den[:N, 0].reshape(N // ROWS, 1, ROWS)

    Wcp = jnp.pad(Wc, ((0, 0), (0, 126)))
    bcp = jnp.pad(bc, (0, 126)).reshape(1, 128)
    out = _k2(h, gcn_sum, agnn_sum, denT, dinvT, jnp.full((1, 1), beta, jnp.float32),
              Wg1, bg1, Wg2, bg2, Wf, bf, Wx, bx, Wcp[:128], Wcp[128:], bcp)
    return out[:, :2]


# speedup vs baseline: 2.4671x; 2.4671x over previous
import functools

import jax
import jax.numpy as jnp
from jax import lax
from jax.experimental import pallas as pl
from jax.experimental.pallas import tpu as pltpu
from jax.experimental.pallas import tpu_sc as plsc

N = 10000
E = 320000
D = 128
H = 128

ROWS = 1000
NTILES = 16
NCORES = 2
EPT = E // NTILES
K = 80
NCHUNK = EPT // K
EPW = E // (NTILES * NCORES)
NP = 10240
RPT = NP // NTILES


def _k1_body(x_ref, w1_ref, b1_ref, w2_ref, b2_ref, w3_ref, b3_ref, h_ref):
    x = x_ref[...]
    h = jnp.maximum(jnp.dot(x, w1_ref[...], preferred_element_type=jnp.float32) + b1_ref[...], 0.0)
    h = jnp.maximum(jnp.dot(h, w2_ref[...], preferred_element_type=jnp.float32) + b2_ref[...], 0.0)
    h_ref[...] = jnp.dot(h, w3_ref[...], preferred_element_type=jnp.float32) + b3_ref[...]


def _k1(x, W1, b1, W2, b2, W3, b3):
    return pl.pallas_call(
        _k1_body,
        grid=(N // ROWS,),
        in_specs=[
            pl.BlockSpec((ROWS, D), lambda i: (i, 0)),
            pl.BlockSpec((D, H), lambda i: (0, 0)),
            pl.BlockSpec((1, H), lambda i: (0, 0)),
            pl.BlockSpec((H, H), lambda i: (0, 0)),
            pl.BlockSpec((1, H), lambda i: (0, 0)),
            pl.BlockSpec((H, H), lambda i: (0, 0)),
            pl.BlockSpec((1, H), lambda i: (0, 0)),
        ],
        out_specs=pl.BlockSpec((ROWS, H), lambda i: (i, 0)),
        out_shape=jax.ShapeDtypeStruct((N, H), jnp.float32),
    )(x, W1, b1.reshape(1, H), W2, b2.reshape(1, H), W3, b3.reshape(1, H))


def _kdeg_body(dst_hbm, onecol_hbm, zeros_hbm, out_hbm, didx, onecol, deg_sh, sem):
    c = lax.axis_index("c")
    t = lax.axis_index("s")
    wid = c * NTILES + t

    pltpu.sync_copy(onecol_hbm, onecol)
    pltpu.sync_copy(zeros_hbm.at[pl.ds(t * RPT, RPT)], deg_sh.at[pl.ds(t * RPT, RPT)])
    plsc.subcore_barrier()

    def chunk(i, _):
        pltpu.sync_copy(dst_hbm.at[pl.ds(wid * EPW + i * K, K)], didx)
        pltpu.sync_copy(onecol, deg_sh.at[didx], add=True)
        return _
    lax.fori_loop(0, EPW // K, chunk, None)

    plsc.subcore_barrier()
    pltpu.sync_copy(deg_sh.at[pl.ds(t * RPT, RPT)], out_hbm.at[c, pl.ds(t * RPT, RPT)])


def _kdeg(dst):
    mesh = plsc.VectorSubcoreMesh(core_axis_name="c", subcore_axis_name="s")
    f = pl.kernel(
        _kdeg_body,
        out_type=jax.ShapeDtypeStruct((NCORES, NP, 16), jnp.float32),
        mesh=mesh,
        compiler_params=pltpu.CompilerParams(use_tc_tiling_on_sc=False, needs_layout_passes=False),
        scratch_types=[
            pltpu.VMEM((K,), jnp.int32),
            pltpu.VMEM((K, 16), jnp.float32),
            pltpu.VMEM_SHARED((NP, 16), jnp.float32),
            pltpu.SemaphoreType.DMA,
        ],
    )
    onecol = jnp.tile(jnp.eye(1, 16, dtype=jnp.float32), (K, 1))
    zeros = jnp.zeros((NP, 16), jnp.float32)
    return f(dst, onecol, zeros)


def _k1b_body(degp_ref, h_ref, dinv_ref, u_ref, hn_ref, inv_ref):
    deg = 1.0 + jnp.sum(degp_ref[0], axis=0)
    dinv = jax.lax.rsqrt(jnp.maximum(deg, 1e-12))
    dinv_ref[...] = dinv.reshape(1, 1, ROWS)
    h = h_ref[...]
    u_ref[...] = dinv[:, None] * h
    n2 = jnp.sum(h * h, axis=1)
    invn = jax.lax.rsqrt(jnp.maximum(n2, 1e-24))
    hn_ref[...] = invn[:, None] * h
    inv_ref[...] = invn.reshape(1, 1, ROWS)


def _k1b(degp_t, h):
    return pl.pallas_call(
        _k1b_body,
        grid=(N // ROWS,),
        in_specs=[
            pl.BlockSpec((1, NCORES, ROWS), lambda i: (i, 0, 0)),
            pl.BlockSpec((ROWS, H), lambda i: (i, 0)),
        ],
        out_specs=[
            pl.BlockSpec((1, 1, ROWS), lambda i: (i, 0, 0)),
            pl.BlockSpec((ROWS, H), lambda i: (i, 0)),
            pl.BlockSpec((ROWS, H), lambda i: (i, 0)),
            pl.BlockSpec((1, 1, ROWS), lambda i: (i, 0, 0)),
        ],
        out_shape=[
            jax.ShapeDtypeStruct((N // ROWS, 1, ROWS), jnp.float32),
            jax.ShapeDtypeStruct((N, H), jnp.float32),
            jax.ShapeDtypeStruct((N, H), jnp.float32),
            jax.ShapeDtypeStruct((N // ROWS, 1, ROWS), jnp.float32),
        ],
    )(degp_t, h)


def _kedge_body(h_hbm, hn_hbm, u_hbm, src_hbm, dst_hbm, beta_hbm,
                zrows_hbm, zden_hbm,
                out_hbm, den_hbm,
                betav, sidx, didx, rows_s, rows_n, pay, svals,
                acc, den_sh, sem_s, sem_d, sem_n):
    role = lax.axis_index("c")
    t = lax.axis_index("s")
    zero16 = jnp.zeros((16,), jnp.float32)
    iota16 = lax.iota(jnp.int32, 16)
    e0mask = (iota16 == 0).astype(jnp.float32)

    pltpu.sync_copy(beta_hbm, betav)
    pltpu.sync_copy(zrows_hbm.at[pl.ds(t * RPT, RPT)], acc.at[pl.ds(t * RPT, RPT)])
    pltpu.sync_copy(zden_hbm.at[pl.ds(t * RPT, RPT)], den_sh.at[pl.ds(t * RPT, RPT)])

    plsc.subcore_barrier()

    def chunk(i, _):
        base = t * EPT + i * K
        pltpu.sync_copy(src_hbm.at[pl.ds(base, K)], sidx)
        pltpu.sync_copy(dst_hbm.at[pl.ds(base, K)], didx)

        @pl.when(role == 0)
        def _r0():
            pltpu.async_copy(u_hbm.at[sidx], rows_s, sem_s).wait()
            pltpu.sync_copy(rows_s, acc.at[didx], add=True)

        @pl.when(role == 1)
        def _r1():
            pltpu.async_copy(h_hbm.at[sidx], rows_s, sem_s).wait()
            pltpu.async_copy(hn_hbm.at[sidx], rows_n, sem_n).wait()
            pltpu.async_copy(hn_hbm.at[didx], pay, sem_d).wait()

            j16s = [jb * 16 + iota16 for jb in range(D // 16)]

            def edge(e, _1):
                e16 = jnp.full((16,), e, jnp.int32)
                q = zero16
                for jb in range(D // 16):
                    vn = plsc.load_gather(rows_n, [e16, j16s[jb]])
                    vd = plsc.load_gather(pay, [e16, j16s[jb]])
                    q = q + vn * vd
                qs = jnp.sum(q)
                s16 = jnp.exp(betav[...] * jnp.full((16,), qs))
                for jb in range(D // 16):
                    va = plsc.load_gather(rows_s, [e16, j16s[jb]])
                    plsc.store_scatter(pay, [e16, j16s[jb]], s16 * va)
                plsc.store_scatter(svals, [e16, iota16], s16 * e0mask)
                return _1

            lax.fori_loop(0, K, edge, None)

            pltpu.sync_copy(pay, acc.at[didx], add=True)
            pltpu.sync_copy(svals, den_sh.at[didx], add=True)
        return _

    lax.fori_loop(0, NCHUNK, chunk, None)

    plsc.subcore_barrier()
    pltpu.sync_copy(acc.at[pl.ds(t * RPT, RPT)], out_hbm.at[role, pl.ds(t * RPT, RPT)])

    @pl.when(role == 1)
    def _fd():
        pltpu.sync_copy(den_sh.at[pl.ds(t * RPT, RPT)], den_hbm.at[pl.ds(t * RPT, RPT)])


def _kedge(h, hn, u, src, dst, beta16):
    mesh = plsc.VectorSubcoreMesh(core_axis_name="c", subcore_axis_name="s")
    f = pl.kernel(
        _kedge_body,
        out_type=[
            jax.ShapeDtypeStruct((NCORES, NP, D), jnp.float32),
            jax.ShapeDtypeStruct((NP, 16), jnp.float32),
        ],
        mesh=mesh,
        compiler_params=pltpu.CompilerParams(use_tc_tiling_on_sc=False, needs_layout_passes=False),
        scratch_types=[
            pltpu.VMEM((16,), jnp.float32),
            pltpu.VMEM((K,), jnp.int32),
            pltpu.VMEM((K,), jnp.int32),
            pltpu.VMEM((K, D), jnp.float32),
            pltpu.VMEM((K, D), jnp.float32),
            pltpu.VMEM((K, D), jnp.float32),
            pltpu.VMEM((K, 16), jnp.float32),
            pltpu.VMEM_SHARED((NP, D), jnp.float32),
            pltpu.VMEM_SHARED((NP, 16), jnp.float32),
            pltpu.SemaphoreType.DMA,
            pltpu.SemaphoreType.DMA,
            pltpu.SemaphoreType.DMA,
        ],
    )
    zrows = jnp.zeros((NP, D), jnp.float32)
    zden = jnp.zeros((NP, 16), jnp.float32)
    return f(h, hn, u, src, dst, beta16, zrows, zden)


def _k2_body(h_ref, gs_ref, as_ref, den_ref, dinv_ref, beta_ref,
             wg1_ref, bg1_ref, wg2_ref, bg2_ref, wf_ref, bf_ref,
             wx_ref, bx_ref, wc1_ref, wc2_ref, bc_ref, out_ref):
    h = h_ref[...]
    dinv = dinv_ref[0, 0, :][:, None]
    beta = beta_ref[0, 0]
    sself = jnp.exp(beta)

    agg = dinv * gs_ref[...] + (dinv * dinv) * h
    h_a = jnp.dot(agg, wg1_ref[...], preferred_element_type=jnp.float32) + bg1_ref[...]
    h_b = jnp.dot(agg, wg2_ref[...], preferred_element_type=jnp.float32) + bg2_ref[...]
    h1 = (as_ref[...] + sself * h) / (den_ref[0, 0, :][:, None] + sself)

    pa = jnp.tanh(jnp.dot(h_a, wf_ref[...], preferred_element_type=jnp.float32) + bf_ref[...])
    pb = jnp.tanh(jnp.dot(h_b, wf_ref[...], preferred_element_type=jnp.float32) + bf_ref[...])
    xp = jnp.tanh(jnp.dot(h, wx_ref[...], preferred_element_type=jnp.float32) + bx_ref[...])
    la = jnp.sum(pa * xp, axis=1)
    lb = jnp.sum(pb * xp, axis=1)
    m = jnp.maximum(la, lb)
    wa = jnp.exp(la - m)
    wb = jnp.exp(lb - m)
    res = (h_a * wa[:, None] + h_b * wb[:, None]) / (wa + wb)[:, None]
    out_ref[...] = (jnp.dot(res, wc1_ref[...], preferred_element_type=jnp.float32)
                    + jnp.dot(h1, wc2_ref[...], preferred_element_type=jnp.float32)
                    + bc_ref[...])


def _k2(h, gcn_sum, agnn_sum, denT, dinvT, beta11,
        Wg1, bg1, Wg2, bg2, Wf, bf, Wx, bx, Wc1p, Wc2p, bcp):
    full = lambda i: (0, 0)
    return pl.pallas_call(
        _k2_body,
        grid=(N // ROWS,),
        in_specs=[
            pl.BlockSpec((ROWS, H), lambda i: (i, 0)),
            pl.BlockSpec((ROWS, H), lambda i: (i, 0)),
            pl.BlockSpec((ROWS, H), lambda i: (i, 0)),
            pl.BlockSpec((1, 1, ROWS), lambda i: (i, 0, 0)),
            pl.BlockSpec((1, 1, ROWS), lambda i: (i, 0, 0)),
            pl.BlockSpec((1, 1), full),
            pl.BlockSpec((H, H), full),
            pl.BlockSpec((1, H), full),
            pl.BlockSpec((H, H), full),
            pl.BlockSpec((1, H), full),
            pl.BlockSpec((H, H), full),
            pl.BlockSpec((1, H), full),
            pl.BlockSpec((H, H), full),
            pl.BlockSpec((1, H), full),
            pl.BlockSpec((H, 128), full),
            pl.BlockSpec((H, 128), full),
            pl.BlockSpec((1, 128), full),
        ],
        out_specs=pl.BlockSpec((ROWS, 128), lambda i: (i, 0)),
        out_shape=jax.ShapeDtypeStruct((N, 128), jnp.float32),
    )(h, gcn_sum, agnn_sum, denT, dinvT, beta11,
      Wg1, bg1.reshape(1, H), Wg2, bg2.reshape(1, H), Wf, bf.reshape(1, H),
      Wx, bx.reshape(1, H), Wc1p, Wc2p, bcp)


def kernel(x, edge_index, W1, b1, W2, b2, W3, b3, Wg1, bg1, Wg2, bg2, beta, Wf, bf, Wx, bx, Wc, bc):
    src = edge_index[0].astype(jnp.int32)
    dst = edge_index[1].astype(jnp.int32)

    h = _k1(x, W1, b1, W2, b2, W3, b3)
    degp = _kdeg(dst)[:, :N, 0]
    degp_t = degp.reshape(NCORES, N // ROWS, ROWS).transpose(1, 0, 2)
    dinvT, u, hn, _invT = _k1b(degp_t, h)

    beta16 = jnp.full((16,), beta, jnp.float32)
    acc, den = _kedge(h, hn, u, src, dst, beta16)

    gcn_sum = acc[0, :N]
    agnn_sum = acc[1, :N]
    denT = den[:N, 0].reshape(N // ROWS, 1, ROWS)

    Wcp = jnp.pad(Wc, ((0, 0), (0, 126)))
    bcp = jnp.pad(bc, (0, 126)).reshape(1, 128)
    out = _k2(h, gcn_sum, agnn_sum, denT, dinvT, jnp.full((1, 1), beta, jnp.float32),
              Wg1, bg1, Wg2, bg2, Wf, bf, Wx, bx, Wcp[:128], Wcp[128:], bcp)
    return out[:, :2]


# speedup vs baseline: 2.8143x; 1.1407x over previous
import functools

import jax
import jax.numpy as jnp
from jax import lax
from jax.experimental import pallas as pl
from jax.experimental.pallas import tpu as pltpu
from jax.experimental.pallas import tpu_sc as plsc

N = 10000
E = 320000
D = 128
H = 128

ROWS = 1000
NTILES = 16
NCORES = 2
EPT = E // NTILES
K = 80
NCHUNK = EPT // K
EPW = E // (NTILES * NCORES)
NP = 10240
RPT = NP // NTILES


def _k1_body(x_ref, w1_ref, b1_ref, w2_ref, b2_ref, w3_ref, b3_ref, h_ref):
    x = x_ref[...]
    h = jnp.maximum(jnp.dot(x, w1_ref[...], preferred_element_type=jnp.float32) + b1_ref[...], 0.0)
    h = jnp.maximum(jnp.dot(h, w2_ref[...], preferred_element_type=jnp.float32) + b2_ref[...], 0.0)
    h_ref[...] = jnp.dot(h, w3_ref[...], preferred_element_type=jnp.float32) + b3_ref[...]


def _k1(x, W1, b1, W2, b2, W3, b3):
    return pl.pallas_call(
        _k1_body,
        grid=(N // ROWS,),
        in_specs=[
            pl.BlockSpec((ROWS, D), lambda i: (i, 0)),
            pl.BlockSpec((D, H), lambda i: (0, 0)),
            pl.BlockSpec((1, H), lambda i: (0, 0)),
            pl.BlockSpec((H, H), lambda i: (0, 0)),
            pl.BlockSpec((1, H), lambda i: (0, 0)),
            pl.BlockSpec((H, H), lambda i: (0, 0)),
            pl.BlockSpec((1, H), lambda i: (0, 0)),
        ],
        out_specs=pl.BlockSpec((ROWS, H), lambda i: (i, 0)),
        out_shape=jax.ShapeDtypeStruct((N, H), jnp.float32),
    )(x, W1, b1.reshape(1, H), W2, b2.reshape(1, H), W3, b3.reshape(1, H))


def _kdeg_body(dst_hbm, onecol_hbm, zeros_hbm, out_hbm, didx, onecol, deg_sh, sem):
    c = lax.axis_index("c")
    t = lax.axis_index("s")
    wid = c * NTILES + t

    pltpu.sync_copy(onecol_hbm, onecol)
    pltpu.sync_copy(zeros_hbm.at[pl.ds(t * RPT, RPT)], deg_sh.at[pl.ds(t * RPT, RPT)])
    plsc.subcore_barrier()

    def chunk(i, _):
        pltpu.sync_copy(dst_hbm.at[pl.ds(wid * EPW + i * K, K)], didx)
        pltpu.sync_copy(onecol, deg_sh.at[didx], add=True)
        return _
    lax.fori_loop(0, EPW // K, chunk, None)

    plsc.subcore_barrier()
    pltpu.sync_copy(deg_sh.at[pl.ds(t * RPT, RPT)], out_hbm.at[c, pl.ds(t * RPT, RPT)])


def _kdeg(dst):
    mesh = plsc.VectorSubcoreMesh(core_axis_name="c", subcore_axis_name="s")
    f = pl.kernel(
        _kdeg_body,
        out_type=jax.ShapeDtypeStruct((NCORES, NP, 16), jnp.float32),
        mesh=mesh,
        compiler_params=pltpu.CompilerParams(use_tc_tiling_on_sc=False, needs_layout_passes=False),
        scratch_types=[
            pltpu.VMEM((K,), jnp.int32),
            pltpu.VMEM((K, 16), jnp.float32),
            pltpu.VMEM_SHARED((NP, 16), jnp.float32),
            pltpu.SemaphoreType.DMA,
        ],
    )
    onecol = jnp.tile(jnp.eye(1, 16, dtype=jnp.float32), (K, 1))
    zeros = jnp.zeros((NP, 16), jnp.float32)
    return f(dst, onecol, zeros)


def _k1b_body(degp_ref, h_ref, dinv_ref, u_ref, hn_ref, inv_ref):
    deg = 1.0 + jnp.sum(degp_ref[0], axis=0)
    dinv = jax.lax.rsqrt(jnp.maximum(deg, 1e-12))
    dinv_ref[...] = dinv.reshape(1, 1, ROWS)
    h = h_ref[...]
    u_ref[...] = dinv[:, None] * h
    n2 = jnp.sum(h * h, axis=1)
    invn = jax.lax.rsqrt(jnp.maximum(n2, 1e-24))
    hn_ref[...] = invn[:, None] * h
    inv_ref[...] = invn.reshape(1, 1, ROWS)


def _k1b(degp_t, h):
    return pl.pallas_call(
        _k1b_body,
        grid=(N // ROWS,),
        in_specs=[
            pl.BlockSpec((1, NCORES, ROWS), lambda i: (i, 0, 0)),
            pl.BlockSpec((ROWS, H), lambda i: (i, 0)),
        ],
        out_specs=[
            pl.BlockSpec((1, 1, ROWS), lambda i: (i, 0, 0)),
            pl.BlockSpec((ROWS, H), lambda i: (i, 0)),
            pl.BlockSpec((ROWS, H), lambda i: (i, 0)),
            pl.BlockSpec((1, 1, ROWS), lambda i: (i, 0, 0)),
        ],
        out_shape=[
            jax.ShapeDtypeStruct((N // ROWS, 1, ROWS), jnp.float32),
            jax.ShapeDtypeStruct((N, H), jnp.float32),
            jax.ShapeDtypeStruct((N, H), jnp.float32),
            jax.ShapeDtypeStruct((N // ROWS, 1, ROWS), jnp.float32),
        ],
    )(degp_t, h)


def _kedge_body(h_hbm, hn_hbm, u_hbm, src_hbm, dst_hbm, beta_hbm,
                zrows_hbm, zden_hbm,
                out_hbm, den_hbm,
                betav, sidx, didx, rows_s, rows_n, pay, svals,
                acc, den_sh, sem_s, sem_d, sem_n):
    role = lax.axis_index("c")
    t = lax.axis_index("s")
    zero16 = jnp.zeros((16,), jnp.float32)
    iota16 = lax.iota(jnp.int32, 16)
    e0mask = (iota16 == 0).astype(jnp.float32)

    pltpu.sync_copy(beta_hbm, betav)
    pltpu.sync_copy(zrows_hbm.at[pl.ds(t * RPT, RPT)], acc.at[pl.ds(t * RPT, RPT)])
    pltpu.sync_copy(zden_hbm.at[pl.ds(t * RPT, RPT)], den_sh.at[pl.ds(t * RPT, RPT)])

    plsc.subcore_barrier()

    def chunk(i, _):
        base = t * EPT + i * K
        pltpu.sync_copy(src_hbm.at[pl.ds(base, K)], sidx)
        pltpu.sync_copy(dst_hbm.at[pl.ds(base, K)], didx)

        @pl.when(role == 0)
        def _r0():
            pltpu.async_copy(u_hbm.at[sidx], rows_s, sem_s).wait()
            pltpu.sync_copy(rows_s, acc.at[didx], add=True)

        @pl.when(role == 1)
        def _r1():
            cp_s = pltpu.async_copy(h_hbm.at[sidx], rows_s, sem_s)
            cp_n = pltpu.async_copy(hn_hbm.at[sidx], rows_n, sem_n)
            cp_d = pltpu.async_copy(hn_hbm.at[didx], pay, sem_d)
            cp_s.wait()
            cp_n.wait()
            cp_d.wait()

            j16s = [jb * 16 + iota16 for jb in range(D // 16)]

            def edge(e, _1):
                e16 = jnp.full((16,), e, jnp.int32)
                q = zero16
                for jb in range(D // 16):
                    vn = plsc.load_gather(rows_n, [e16, j16s[jb]])
                    vd = plsc.load_gather(pay, [e16, j16s[jb]])
                    q = q + vn * vd
                qs = jnp.sum(q)
                s16 = jnp.exp(betav[...] * jnp.full((16,), qs))
                for jb in range(D // 16):
                    va = plsc.load_gather(rows_s, [e16, j16s[jb]])
                    plsc.store_scatter(pay, [e16, j16s[jb]], s16 * va)
                plsc.store_scatter(svals, [e16, iota16], s16 * e0mask)
                return _1

            lax.fori_loop(0, K, edge, None)

            pltpu.sync_copy(pay, acc.at[didx], add=True)
            pltpu.sync_copy(svals, den_sh.at[didx], add=True)
        return _

    lax.fori_loop(0, NCHUNK, chunk, None)

    plsc.subcore_barrier()
    pltpu.sync_copy(acc.at[pl.ds(t * RPT, RPT)], out_hbm.at[role, pl.ds(t * RPT, RPT)])

    @pl.when(role == 1)
    def _fd():
        pltpu.sync_copy(den_sh.at[pl.ds(t * RPT, RPT)], den_hbm.at[pl.ds(t * RPT, RPT)])


def _kedge(h, hn, u, src, dst, beta16):
    mesh = plsc.VectorSubcoreMesh(core_axis_name="c", subcore_axis_name="s")
    f = pl.kernel(
        _kedge_body,
        out_type=[
            jax.ShapeDtypeStruct((NCORES, NP, D), jnp.float32),
            jax.ShapeDtypeStruct((NP, 16), jnp.float32),
        ],
        mesh=mesh,
        compiler_params=pltpu.CompilerParams(use_tc_tiling_on_sc=False, needs_layout_passes=False),
        scratch_types=[
            pltpu.VMEM((16,), jnp.float32),
            pltpu.VMEM((K,), jnp.int32),
            pltpu.VMEM((K,), jnp.int32),
            pltpu.VMEM((K, D), jnp.float32),
            pltpu.VMEM((K, D), jnp.float32),
            pltpu.VMEM((K, D), jnp.float32),
            pltpu.VMEM((K, 16), jnp.float32),
            pltpu.VMEM_SHARED((NP, D), jnp.float32),
            pltpu.VMEM_SHARED((NP, 16), jnp.float32),
            pltpu.SemaphoreType.DMA,
            pltpu.SemaphoreType.DMA,
            pltpu.SemaphoreType.DMA,
        ],
    )
    zrows = jnp.zeros((NP, D), jnp.float32)
    zden = jnp.zeros((NP, 16), jnp.float32)
    return f(h, hn, u, src, dst, beta16, zrows, zden)


def _k2_body(h_ref, gs_ref, as_ref, den_ref, dinv_ref, beta_ref,
             wg1_ref, bg1_ref, wg2_ref, bg2_ref, wf_ref, bf_ref,
             wx_ref, bx_ref, wc1_ref, wc2_ref, bc_ref, out_ref):
    h = h_ref[...]
    dinv = dinv_ref[0, 0, :][:, None]
    beta = beta_ref[0, 0]
    sself = jnp.exp(beta)

    agg = dinv * gs_ref[...] + (dinv * dinv) * h
    h_a = jnp.dot(agg, wg1_ref[...], preferred_element_type=jnp.float32) + bg1_ref[...]
    h_b = jnp.dot(agg, wg2_ref[...], preferred_element_type=jnp.float32) + bg2_ref[...]
    h1 = (as_ref[...] + sself * h) / (den_ref[0, 0, :][:, None] + sself)

    pa = jnp.tanh(jnp.dot(h_a, wf_ref[...], preferred_element_type=jnp.float32) + bf_ref[...])
    pb = jnp.tanh(jnp.dot(h_b, wf_ref[...], preferred_element_type=jnp.float32) + bf_ref[...])
    xp = jnp.tanh(jnp.dot(h, wx_ref[...], preferred_element_type=jnp.float32) + bx_ref[...])
    la = jnp.sum(pa * xp, axis=1)
    lb = jnp.sum(pb * xp, axis=1)
    m = jnp.maximum(la, lb)
    wa = jnp.exp(la - m)
    wb = jnp.exp(lb - m)
    res = (h_a * wa[:, None] + h_b * wb[:, None]) / (wa + wb)[:, None]
    out_ref[...] = (jnp.dot(res, wc1_ref[...], preferred_element_type=jnp.float32)
                    + jnp.dot(h1, wc2_ref[...], preferred_element_type=jnp.float32)
                    + bc_ref[...])


def _k2(h, gcn_sum, agnn_sum, denT, dinvT, beta11,
        Wg1, bg1, Wg2, bg2, Wf, bf, Wx, bx, Wc1p, Wc2p, bcp):
    full = lambda i: (0, 0)
    return pl.pallas_call(
        _k2_body,
        grid=(N // ROWS,),
        in_specs=[
            pl.BlockSpec((ROWS, H), lambda i: (i, 0)),
            pl.BlockSpec((ROWS, H), lambda i: (i, 0)),
            pl.BlockSpec((ROWS, H), lambda i: (i, 0)),
            pl.BlockSpec((1, 1, ROWS), lambda i: (i, 0, 0)),
            pl.BlockSpec((1, 1, ROWS), lambda i: (i, 0, 0)),
            pl.BlockSpec((1, 1), full),
            pl.BlockSpec((H, H), full),
            pl.BlockSpec((1, H), full),
            pl.BlockSpec((H, H), full),
            pl.BlockSpec((1, H), full),
            pl.BlockSpec((H, H), full),
            pl.BlockSpec((1, H), full),
            pl.BlockSpec((H, H), full),
            pl.BlockSpec((1, H), full),
            pl.BlockSpec((H, 128), full),
            pl.BlockSpec((H, 128), full),
            pl.BlockSpec((1, 128), full),
        ],
        out_specs=pl.BlockSpec((ROWS, 128), lambda i: (i, 0)),
        out_shape=jax.ShapeDtypeStruct((N, 128), jnp.float32),
    )(h, gcn_sum, agnn_sum, denT, dinvT, beta11,
      Wg1, bg1.reshape(1, H), Wg2, bg2.reshape(1, H), Wf, bf.reshape(1, H),
      Wx, bx.reshape(1, H), Wc1p, Wc2p, bcp)


def kernel(x, edge_index, W1, b1, W2, b2, W3, b3, Wg1, bg1, Wg2, bg2, beta, Wf, bf, Wx, bx, Wc, bc):
    src = edge_index[0].astype(jnp.int32)
    dst = edge_index[1].astype(jnp.int32)

    h = _k1(x, W1, b1, W2, b2, W3, b3)
    degp = _kdeg(dst)[:, :N, 0]
    degp_t = degp.reshape(NCORES, N // ROWS, ROWS).transpose(1, 0, 2)
    dinvT, u, hn, _invT = _k1b(degp_t, h)

    beta16 = jnp.full((16,), beta, jnp.float32)
    acc, den = _kedge(h, hn, u, src, dst, beta16)

    gcn_sum = acc[0, :N]
    agnn_sum = acc[1, :N]
    denT = den[:N, 0].reshape(N // ROWS, 1, ROWS)

    Wcp = jnp.pad(Wc, ((0, 0), (0, 126)))
    bcp = jnp.pad(bc, (0, 126)).reshape(1, 128)
    out = _k2(h, gcn_sum, agnn_sum, denT, dinvT, jnp.full((1, 1), beta, jnp.float32),
              Wg1, bg1, Wg2, bg2, Wf, bf, Wx, bx, Wcp[:128], Wcp[128:], bcp)
    return out[:, :2]


# speedup vs baseline: 2.8721x; 1.0206x over previous
import functools

import jax
import jax.numpy as jnp
from jax import lax
from jax.experimental import pallas as pl
from jax.experimental.pallas import tpu as pltpu
from jax.experimental.pallas import tpu_sc as plsc

N = 10000
E = 320000
D = 128
H = 128

ROWS = 1000
NTILES = 16
NCORES = 2
EPT = E // NTILES
K = 80
NCHUNK = EPT // K
EPW = E // (NTILES * NCORES)
NP = 10240
RPT = NP // NTILES


def _k1_body(x_ref, w1_ref, b1_ref, w2_ref, b2_ref, w3_ref, b3_ref, h_ref):
    x = x_ref[...]
    h = jnp.maximum(jnp.dot(x, w1_ref[...], preferred_element_type=jnp.float32) + b1_ref[...], 0.0)
    h = jnp.maximum(jnp.dot(h, w2_ref[...], preferred_element_type=jnp.float32) + b2_ref[...], 0.0)
    h_ref[...] = jnp.dot(h, w3_ref[...], preferred_element_type=jnp.float32) + b3_ref[...]


def _k1(x, W1, b1, W2, b2, W3, b3):
    return pl.pallas_call(
        _k1_body,
        grid=(N // ROWS,),
        in_specs=[
            pl.BlockSpec((ROWS, D), lambda i: (i, 0)),
            pl.BlockSpec((D, H), lambda i: (0, 0)),
            pl.BlockSpec((1, H), lambda i: (0, 0)),
            pl.BlockSpec((H, H), lambda i: (0, 0)),
            pl.BlockSpec((1, H), lambda i: (0, 0)),
            pl.BlockSpec((H, H), lambda i: (0, 0)),
            pl.BlockSpec((1, H), lambda i: (0, 0)),
        ],
        out_specs=pl.BlockSpec((ROWS, H), lambda i: (i, 0)),
        out_shape=jax.ShapeDtypeStruct((N, H), jnp.float32),
    )(x, W1, b1.reshape(1, H), W2, b2.reshape(1, H), W3, b3.reshape(1, H))


def _kdeg_body(dst_hbm, onecol_hbm, zeros_hbm, out_hbm, didx, onecol, deg_sh, sem):
    c = lax.axis_index("c")
    t = lax.axis_index("s")
    wid = c * NTILES + t

    pltpu.sync_copy(onecol_hbm, onecol)
    pltpu.sync_copy(zeros_hbm.at[pl.ds(t * RPT, RPT)], deg_sh.at[pl.ds(t * RPT, RPT)])
    plsc.subcore_barrier()

    def chunk(i, _):
        pltpu.sync_copy(dst_hbm.at[pl.ds(wid * EPW + i * K, K)], didx)
        pltpu.sync_copy(onecol, deg_sh.at[didx], add=True)
        return _
    lax.fori_loop(0, EPW // K, chunk, None)

    plsc.subcore_barrier()
    pltpu.sync_copy(deg_sh.at[pl.ds(t * RPT, RPT)], out_hbm.at[c, pl.ds(t * RPT, RPT)])


def _kdeg(dst):
    mesh = plsc.VectorSubcoreMesh(core_axis_name="c", subcore_axis_name="s")
    f = pl.kernel(
        _kdeg_body,
        out_type=jax.ShapeDtypeStruct((NCORES, NP, 16), jnp.float32),
        mesh=mesh,
        compiler_params=pltpu.CompilerParams(use_tc_tiling_on_sc=False, needs_layout_passes=False),
        scratch_types=[
            pltpu.VMEM((K,), jnp.int32),
            pltpu.VMEM((K, 16), jnp.float32),
            pltpu.VMEM_SHARED((NP, 16), jnp.float32),
            pltpu.SemaphoreType.DMA,
        ],
    )
    onecol = jnp.tile(jnp.eye(1, 16, dtype=jnp.float32), (K, 1))
    zeros = jnp.zeros((NP, 16), jnp.float32)
    return f(dst, onecol, zeros)


def _k1b_body(degp_ref, h_ref, dinv_ref, u_ref, hn_ref, inv_ref):
    deg = 1.0 + jnp.sum(degp_ref[0], axis=0)
    dinv = jax.lax.rsqrt(jnp.maximum(deg, 1e-12))
    dinv_ref[...] = dinv.reshape(1, 1, ROWS)
    h = h_ref[...]
    u_ref[...] = dinv[:, None] * h
    n2 = jnp.sum(h * h, axis=1)
    invn = jax.lax.rsqrt(jnp.maximum(n2, 1e-24))
    hn_ref[...] = invn[:, None] * h
    inv_ref[...] = invn.reshape(1, 1, ROWS)


def _k1b(degp_t, h):
    return pl.pallas_call(
        _k1b_body,
        grid=(N // ROWS,),
        in_specs=[
            pl.BlockSpec((1, NCORES, ROWS), lambda i: (i, 0, 0)),
            pl.BlockSpec((ROWS, H), lambda i: (i, 0)),
        ],
        out_specs=[
            pl.BlockSpec((1, 1, ROWS), lambda i: (i, 0, 0)),
            pl.BlockSpec((ROWS, H), lambda i: (i, 0)),
            pl.BlockSpec((ROWS, H), lambda i: (i, 0)),
            pl.BlockSpec((1, 1, ROWS), lambda i: (i, 0, 0)),
        ],
        out_shape=[
            jax.ShapeDtypeStruct((N // ROWS, 1, ROWS), jnp.float32),
            jax.ShapeDtypeStruct((N, H), jnp.float32),
            jax.ShapeDtypeStruct((N, H), jnp.float32),
            jax.ShapeDtypeStruct((N // ROWS, 1, ROWS), jnp.float32),
        ],
    )(degp_t, h)


def _kedge_body(h_hbm, hn_hbm, u_hbm, src_hbm, dst_hbm, beta_hbm,
                zrows_hbm, zden_hbm,
                out_hbm, den_hbm,
                betav, sidx, didx, rows_s, rows_n, pay, svals,
                acc, den_sh, sem_s, sem_d, sem_n, sem_s2, sem_d2, sem_n2):
    role = lax.axis_index("c")
    t = lax.axis_index("s")
    zero16 = jnp.zeros((16,), jnp.float32)
    iota16 = lax.iota(jnp.int32, 16)
    e0mask = (iota16 == 0).astype(jnp.float32)

    pltpu.sync_copy(beta_hbm, betav)
    pltpu.sync_copy(zrows_hbm.at[pl.ds(t * RPT, RPT)], acc.at[pl.ds(t * RPT, RPT)])
    pltpu.sync_copy(zden_hbm.at[pl.ds(t * RPT, RPT)], den_sh.at[pl.ds(t * RPT, RPT)])

    plsc.subcore_barrier()

    def chunk(i, _):
        base = t * EPT + i * K
        pltpu.sync_copy(src_hbm.at[pl.ds(base, K)], sidx)
        pltpu.sync_copy(dst_hbm.at[pl.ds(base, K)], didx)

        @pl.when(role == 0)
        def _r0():
            pltpu.async_copy(u_hbm.at[sidx], rows_s, sem_s).wait()
            pltpu.sync_copy(rows_s, acc.at[didx], add=True)

        @pl.when(role == 1)
        def _r1():
            KH = K // 2
            cps = []
            for lo, sem3 in ((0, (sem_s, sem_n, sem_d)),
                             (KH, (sem_s2, sem_n2, sem_d2))):
                sl = pl.ds(lo, KH)
                cps.append((
                    pltpu.async_copy(h_hbm.at[sidx.at[sl]], rows_s.at[sl], sem3[0]),
                    pltpu.async_copy(hn_hbm.at[sidx.at[sl]], rows_n.at[sl], sem3[1]),
                    pltpu.async_copy(hn_hbm.at[didx.at[sl]], pay.at[sl], sem3[2]),
                ))

            j16s = [jb * 16 + iota16 for jb in range(D // 16)]

            def edge(e, _1):
                e16 = jnp.full((16,), e, jnp.int32)
                q = zero16
                for jb in range(D // 16):
                    vn = plsc.load_gather(rows_n, [e16, j16s[jb]])
                    vd = plsc.load_gather(pay, [e16, j16s[jb]])
                    q = q + vn * vd
                qs = jnp.sum(q)
                s16 = jnp.exp(betav[...] * jnp.full((16,), qs))
                for jb in range(D // 16):
                    va = plsc.load_gather(rows_s, [e16, j16s[jb]])
                    plsc.store_scatter(pay, [e16, j16s[jb]], s16 * va)
                plsc.store_scatter(svals, [e16, iota16], s16 * e0mask)
                return _1

            for half in range(2):
                for cp in cps[half]:
                    cp.wait()
                lax.fori_loop(half * KH, (half + 1) * KH, edge, None)

            pltpu.sync_copy(pay, acc.at[didx], add=True)
            pltpu.sync_copy(svals, den_sh.at[didx], add=True)
        return _

    lax.fori_loop(0, NCHUNK, chunk, None)

    plsc.subcore_barrier()
    pltpu.sync_copy(acc.at[pl.ds(t * RPT, RPT)], out_hbm.at[role, pl.ds(t * RPT, RPT)])

    @pl.when(role == 1)
    def _fd():
        pltpu.sync_copy(den_sh.at[pl.ds(t * RPT, RPT)], den_hbm.at[pl.ds(t * RPT, RPT)])


def _kedge(h, hn, u, src, dst, beta16):
    mesh = plsc.VectorSubcoreMesh(core_axis_name="c", subcore_axis_name="s")
    f = pl.kernel(
        _kedge_body,
        out_type=[
            jax.ShapeDtypeStruct((NCORES, NP, D), jnp.float32),
            jax.ShapeDtypeStruct((NP, 16), jnp.float32),
        ],
        mesh=mesh,
        compiler_params=pltpu.CompilerParams(use_tc_tiling_on_sc=False, needs_layout_passes=False),
        scratch_types=[
            pltpu.VMEM((16,), jnp.float32),
            pltpu.VMEM((K,), jnp.int32),
            pltpu.VMEM((K,), jnp.int32),
            pltpu.VMEM((K, D), jnp.float32),
            pltpu.VMEM((K, D), jnp.float32),
            pltpu.VMEM((K, D), jnp.float32),
            pltpu.VMEM((K, 16), jnp.float32),
            pltpu.VMEM_SHARED((NP, D), jnp.float32),
            pltpu.VMEM_SHARED((NP, 16), jnp.float32),
            pltpu.SemaphoreType.DMA,
            pltpu.SemaphoreType.DMA,
            pltpu.SemaphoreType.DMA,
            pltpu.SemaphoreType.DMA,
            pltpu.SemaphoreType.DMA,
            pltpu.SemaphoreType.DMA,
        ],
    )
    zrows = jnp.zeros((NP, D), jnp.float32)
    zden = jnp.zeros((NP, 16), jnp.float32)
    return f(h, hn, u, src, dst, beta16, zrows, zden)


def _k2_body(h_ref, gs_ref, as_ref, den_ref, dinv_ref, beta_ref,
             wg1_ref, bg1_ref, wg2_ref, bg2_ref, wf_ref, bf_ref,
             wx_ref, bx_ref, wc1_ref, wc2_ref, bc_ref, out_ref):
    h = h_ref[...]
    dinv = dinv_ref[0, 0, :][:, None]
    beta = beta_ref[0, 0]
    sself = jnp.exp(beta)

    agg = dinv * gs_ref[...] + (dinv * dinv) * h
    h_a = jnp.dot(agg, wg1_ref[...], preferred_element_type=jnp.float32) + bg1_ref[...]
    h_b = jnp.dot(agg, wg2_ref[...], preferred_element_type=jnp.float32) + bg2_ref[...]
    h1 = (as_ref[...] + sself * h) / (den_ref[0, 0, :][:, None] + sself)

    pa = jnp.tanh(jnp.dot(h_a, wf_ref[...], preferred_element_type=jnp.float32) + bf_ref[...])
    pb = jnp.tanh(jnp.dot(h_b, wf_ref[...], preferred_element_type=jnp.float32) + bf_ref[...])
    xp = jnp.tanh(jnp.dot(h, wx_ref[...], preferred_element_type=jnp.float32) + bx_ref[...])
    la = jnp.sum(pa * xp, axis=1)
    lb = jnp.sum(pb * xp, axis=1)
    m = jnp.maximum(la, lb)
    wa = jnp.exp(la - m)
    wb = jnp.exp(lb - m)
    res = (h_a * wa[:, None] + h_b * wb[:, None]) / (wa + wb)[:, None]
    out_ref[...] = (jnp.dot(res, wc1_ref[...], preferred_element_type=jnp.float32)
                    + jnp.dot(h1, wc2_ref[...], preferred_element_type=jnp.float32)
                    + bc_ref[...])


def _k2(h, gcn_sum, agnn_sum, denT, dinvT, beta11,
        Wg1, bg1, Wg2, bg2, Wf, bf, Wx, bx, Wc1p, Wc2p, bcp):
    full = lambda i: (0, 0)
    return pl.pallas_call(
        _k2_body,
        grid=(N // ROWS,),
        in_specs=[
            pl.BlockSpec((ROWS, H), lambda i: (i, 0)),
            pl.BlockSpec((ROWS, H), lambda i: (i, 0)),
            pl.BlockSpec((ROWS, H), lambda i: (i, 0)),
            pl.BlockSpec((1, 1, ROWS), lambda i: (i, 0, 0)),
            pl.BlockSpec((1, 1, ROWS), lambda i: (i, 0, 0)),
            pl.BlockSpec((1, 1), full),
            pl.BlockSpec((H, H), full),
            pl.BlockSpec((1, H), full),
            pl.BlockSpec((H, H), full),
            pl.BlockSpec((1, H), full),
            pl.BlockSpec((H, H), full),
            pl.BlockSpec((1, H), full),
            pl.BlockSpec((H, H), full),
            pl.BlockSpec((1, H), full),
            pl.BlockSpec((H, 128), full),
            pl.BlockSpec((H, 128), full),
            pl.BlockSpec((1, 128), full),
        ],
        out_specs=pl.BlockSpec((ROWS, 128), lambda i: (i, 0)),
        out_shape=jax.ShapeDtypeStruct((N, 128), jnp.float32),
    )(h, gcn_sum, agnn_sum, denT, dinvT, beta11,
      Wg1, bg1.reshape(1, H), Wg2, bg2.reshape(1, H), Wf, bf.reshape(1, H),
      Wx, bx.reshape(1, H), Wc1p, Wc2p, bcp)


def kernel(x, edge_index, W1, b1, W2, b2, W3, b3, Wg1, bg1, Wg2, bg2, beta, Wf, bf, Wx, bx, Wc, bc):
    src = edge_index[0].astype(jnp.int32)
    dst = edge_index[1].astype(jnp.int32)

    h = _k1(x, W1, b1, W2, b2, W3, b3)
    degp = _kdeg(dst)[:, :N, 0]
    degp_t = degp.reshape(NCORES, N // ROWS, ROWS).transpose(1, 0, 2)
    dinvT, u, hn, _invT = _k1b(degp_t, h)

    beta16 = jnp.full((16,), beta, jnp.float32)
    acc, den = _kedge(h, hn, u, src, dst, beta16)

    gcn_sum = acc[0, :N]
    agnn_sum = acc[1, :N]
    denT = den[:N, 0].reshape(N // ROWS, 1, ROWS)

    Wcp = jnp.pad(Wc, ((0, 0), (0, 126)))
    bcp = jnp.pad(bc, (0, 126)).reshape(1, 128)
    out = _k2(h, gcn_sum, agnn_sum, denT, dinvT, jnp.full((1, 1), beta, jnp.float32),
              Wg1, bg1, Wg2, bg2, Wf, bf, Wx, bx, Wcp[:128], Wcp[128:], bcp)
    return out[:, :2]
